# baseline (speedup 1.0000x reference)
import functools

import jax
import jax.numpy as jnp
from jax import lax
from jax.experimental import pallas as pl
from jax.experimental.pallas import tpu as pltpu
from jax.experimental.pallas import tpu_sc as plsc

N = 10000
DEG = 16
E = 160000
EDEG = 4
DN = 128
DE = 16
H = 128

NTILES = 32

ECHUNK = 1000
E_CHUNKS_PER_TILE = E // ECHUNK // NTILES
E8 = E // 8

NCHUNK = 256
N_PAD = 10240
N_CHUNKS = N_PAD // NCHUNK

_mesh = plsc.VectorSubcoreMesh(core_axis_name="c", subcore_axis_name="s")


def _wid():
    return lax.axis_index("c") * 16 + lax.axis_index("s")


def _edge_gather_body(ef_hbm, enf_hbm, out_hbm, idxs, buf_v, own_v, acc_v,
                      isem, gsem, osem, wsem):
    wid = _wid()
    nc = E_CHUNKS_PER_TILE

    for c in range(nc):
        base = (wid * nc + c) * ECHUNK
        pltpu.async_copy(enf_hbm.at[pl.ds(base * EDEG, ECHUNK * EDEG)],
                         idxs[c], isem)

    def chunk_body(c, _):
        base = (wid * nc + c) * ECHUNK

        pltpu.make_async_copy(
            enf_hbm.at[pl.ds(0, ECHUNK * EDEG)], idxs[0], isem).wait()
        down = pltpu.async_copy(ef_hbm.at[pl.ds(base, ECHUNK)], own_v, wsem)
        for c2 in range(nc):
            @pl.when(c2 == c)
            def _fire(c2=c2):
                pltpu.async_copy(ef_hbm.at[idxs[c2]], buf_v, gsem)
        pltpu.make_async_copy(ef_hbm.at[idxs[0]], buf_v, gsem).wait()
        down.wait()

        def row_body(r, _):
            for u in range(8):
                e = r * 8 + u
                s = pl.ds(u * DE, DE)
                acc_v[r, s] = (own_v[e, :] + buf_v[4 * e, :]
                               + buf_v[4 * e + 1, :] + buf_v[4 * e + 2, :]
                               + buf_v[4 * e + 3, :])
            return _

        lax.fori_loop(0, ECHUNK // 8, row_body, None)
        pltpu.async_copy(
            acc_v, out_hbm.at[pl.ds((wid * nc + c) * (ECHUNK // 8),
                                    ECHUNK // 8)], osem)
        pltpu.make_async_copy(
            acc_v, out_hbm.at[pl.ds(0, ECHUNK // 8)], osem).wait()
        return _

    lax.fori_loop(0, nc, chunk_body, None)


@functools.partial(
    pl.kernel,
    out_type=jax.ShapeDtypeStruct((E8, 128), jnp.float32),
    mesh=_mesh,
    scratch_types=[
        [pltpu.VMEM((ECHUNK * EDEG,), jnp.int32)
         for _ in range(E_CHUNKS_PER_TILE)],
        pltpu.VMEM((ECHUNK * EDEG, DE), jnp.float32),
        pltpu.VMEM((ECHUNK, DE), jnp.float32),
        pltpu.VMEM((ECHUNK // 8, 128), jnp.float32),
        pltpu.SemaphoreType.DMA,
        pltpu.SemaphoreType.DMA,
        pltpu.SemaphoreType.DMA,
        pltpu.SemaphoreType.DMA,
    ],
    compiler_params=pltpu.CompilerParams(use_tc_tiling_on_sc=False),
)
def _edge_gather(ef_hbm, enf_hbm, out_hbm, idxs, buf_v, own_v, acc_v,
                 isem, gsem, osem, wsem):
    _edge_gather_body(ef_hbm, enf_hbm, out_hbm, idxs, buf_v, own_v, acc_v,
                      isem, gsem, osem, wsem)


NWN = (DEG + 1) * NCHUNK
NWE = DEG * NCHUNK


def _node_fire_idx(nnt, net, idxn, idxe, isem, c):
    pltpu.async_copy(nnt.at[pl.ds(c * NWN, NWN)], idxn, isem)
    pltpu.async_copy(net.at[pl.ds(c * NWE, NWE)], idxe, isem)


def _node_drain_idx(nnt, net, idxn, idxe, isem):
    pltpu.make_async_copy(nnt.at[pl.ds(0, NWN)], idxn, isem).wait()
    pltpu.make_async_copy(net.at[pl.ds(0, NWE)], idxe, isem).wait()


def _node_zero(accn, acce):
    zero16 = jnp.zeros((16,), jnp.float32)

    def zrow(r, _):
        for u in range(H // 16):
            s = pl.ds(u * 16, 16)
            accn[r, s] = zero16
            acce[r, s] = zero16
        return _

    lax.fori_loop(0, NCHUNK, zrow, None)


def _node_fire_adds(nf, ne_tab, idxn, idxe, accn, acce, gsem):
    for j in range(DEG + 1):
        pltpu.async_copy(nf.at[idxn.at[pl.ds(j * NCHUNK, NCHUNK)]], accn,
                         gsem, add=True)
    for j in range(DEG):
        pltpu.async_copy(ne_tab.at[idxe.at[pl.ds(j * NCHUNK, NCHUNK)]], acce,
                         gsem, add=True)


def _node_drain_adds(nf, ne_tab, idxn, idxe, accn, acce, gsem):
    for _ in range(DEG + 1):
        pltpu.make_async_copy(
            nf.at[idxn.at[pl.ds(0, NCHUNK)]], accn, gsem).wait()
    for _ in range(DEG):
        pltpu.make_async_copy(
            ne_tab.at[idxe.at[pl.ds(0, NCHUNK)]], acce, gsem).wait()


def _node_fire_outs(accn, acce, outn, oute, osem, base):
    pltpu.async_copy(accn, outn.at[pl.ds(base, NCHUNK)], osem)
    pltpu.async_copy(acce, oute.at[pl.ds(base, NCHUNK)], osem)


def _node_drain_outs(accn, acce, outn, oute, osem):
    pltpu.make_async_copy(accn, outn.at[pl.ds(0, NCHUNK)], osem).wait()
    pltpu.make_async_copy(acce, oute.at[pl.ds(0, NCHUNK)], osem).wait()


def _node_gather_body(nf, ne_tab, nnt, net, outn, oute,
                      idxns, idxes, accn, acce, isems, gsem, osem):
    wid = _wid()
    c0 = wid
    c1 = 32 + (wid // 16) * 4 + (wid % 16)
    has2 = (wid % 16) < 4

    _node_fire_idx(nnt, net, idxns[0], idxes[0], isems[0], c0)

    @pl.when(has2)
    def _idx1():
        _node_fire_idx(nnt, net, idxns[1], idxes[1], isems[1], c1)

    _node_zero(accn, acce)
    _node_drain_idx(nnt, net, idxns[0], idxes[0], isems[0])
    _node_fire_adds(nf, ne_tab, idxns[0], idxes[0], accn, acce, gsem)
    _node_drain_adds(nf, ne_tab, idxns[0], idxes[0], accn, acce, gsem)
    _node_fire_outs(accn, acce, outn, oute, osem, c0 * NCHUNK)

    @pl.when(has2)
    def _chunk1():
        _node_drain_outs(accn, acce, outn, oute, osem)
        _node_zero(accn, acce)
        _node_drain_idx(nnt, net, idxns[1], idxes[1], isems[1])
        _node_fire_adds(nf, ne_tab, idxns[1], idxes[1], accn, acce, gsem)
        _node_drain_adds(nf, ne_tab, idxns[1], idxes[1], accn, acce, gsem)
        _node_fire_outs(accn, acce, outn, oute, osem, c1 * NCHUNK)

    _node_drain_outs(accn, acce, outn, oute, osem)


@functools.partial(
    pl.kernel,
    out_type=(jax.ShapeDtypeStruct((N_PAD, DN), jnp.float32),
              jax.ShapeDtypeStruct((N_PAD, H), jnp.float32)),
    mesh=_mesh,
    scratch_types=[
        [pltpu.VMEM((NWN,), jnp.int32) for _ in range(2)],
        [pltpu.VMEM((NWE,), jnp.int32) for _ in range(2)],
        pltpu.VMEM((NCHUNK, DN), jnp.float32),
        pltpu.VMEM((NCHUNK, H), jnp.float32),
        [pltpu.SemaphoreType.DMA for _ in range(2)],
        pltpu.SemaphoreType.DMA,
        pltpu.SemaphoreType.DMA,
    ],
)
def _node_gather(nf, ne_tab, nnt, net, outn, oute,
                 idxns, idxes, accn, acce, isems, gsem, osem):
    _node_gather_body(nf, ne_tab, nnt, net, outn, oute,
                      idxns, idxes, accn, acce, isems, gsem, osem)


def _edge_mlp_kernel(x8_ref, w1_ref, b1_ref, w2_ref, b2_ref, g_ref, b_ref,
                     o_ref):
    w1 = w1_ref[...]
    w2 = w2_ref[...]
    b1 = float(EDEG + 1) * b1_ref[...]
    b2 = b2_ref[...]
    g = g_ref[...]
    b = b_ref[...]
    for u in range(8):
        x = x8_ref[:, u * DE:(u + 1) * DE]
        h1 = jnp.dot(x, w1, preferred_element_type=jnp.float32) + b1
        h2 = jax.nn.relu(jnp.dot(h1, w2,
                                 preferred_element_type=jnp.float32) + b2)
        m = jnp.mean(h2, axis=-1, keepdims=True)
        v = jnp.mean((h2 - m) ** 2, axis=-1, keepdims=True)
        o_ref[:, u, :] = (h2 - m) / jnp.sqrt(v + 1e-5) * g + b


def _edge_mlp(agg8, WeT, be2, WceT, bce2, g2, b2):
    blk = 1000
    grid = E8 // blk
    out3 = pl.pallas_call(
        _edge_mlp_kernel,
        grid=(grid,),
        in_specs=[
            pl.BlockSpec((blk, 128), lambda i: (i, 0)),
            pl.BlockSpec((DE, H), lambda i: (0, 0)),
            pl.BlockSpec((1, H), lambda i: (0, 0)),
            pl.BlockSpec((H, H), lambda i: (0, 0)),
            pl.BlockSpec((1, H), lambda i: (0, 0)),
            pl.BlockSpec((1, H), lambda i: (0, 0)),
            pl.BlockSpec((1, H), lambda i: (0, 0)),
        ],
        out_specs=pl.BlockSpec((blk, 8, H), lambda i: (i, 0, 0)),
        out_shape=jax.ShapeDtypeStruct((E8, 8, H), jnp.float32),
    )(agg8, WeT, be2, WceT, bce2, g2, b2)
    return out3.reshape(E, H)


def _node_mlp_kernel(x_ref, e_ref, w1_ref, b1_ref, w2_ref, b2_ref, g_ref,
                     b_ref, o_ref):
    h1 = (jnp.dot(x_ref[...], w1_ref[...], preferred_element_type=jnp.float32)
          + float(DEG + 1) * b1_ref[...] + e_ref[...])
    h2 = jax.nn.relu(jnp.dot(h1, w2_ref[...],
                             preferred_element_type=jnp.float32) + b2_ref[...])
    m = jnp.mean(h2, axis=-1, keepdims=True)
    v = jnp.mean((h2 - m) ** 2, axis=-1, keepdims=True)
    o_ref[...] = (h2 - m) / jnp.sqrt(v + 1e-5) * g_ref[...] + b_ref[...]


def _node_mlp(agg_nf, agg_ne, WnT, bn2, WcnT, bcn2, g2, b2):
    blk = 1000
    grid = N // blk
    return pl.pallas_call(
        _node_mlp_kernel,
        grid=(grid,),
        in_specs=[
            pl.BlockSpec((blk, DN), lambda i: (i, 0)),
            pl.BlockSpec((blk, H), lambda i: (i, 0)),
            pl.BlockSpec((DN, H), lambda i: (0, 0)),
            pl.BlockSpec((1, H), lambda i: (0, 0)),
            pl.BlockSpec((H, H), lambda i: (0, 0)),
            pl.BlockSpec((1, H), lambda i: (0, 0)),
            pl.BlockSpec((1, H), lambda i: (0, 0)),
            pl.BlockSpec((1, H), lambda i: (0, 0)),
        ],
        out_specs=pl.BlockSpec((blk, H), lambda i: (i, 0)),
        out_shape=jax.ShapeDtypeStruct((N, H), jnp.float32),
    )(agg_nf, agg_ne, WnT, bn2, WcnT, bcn2, g2, b2)


def kernel(node_neighbors, edge_neighbors, node_feats, edge_feats,
           We, be, Wn, bn, Wce, bce, Wcn, bcn, ln_g, ln_b):
    en_flat = edge_neighbors.astype(jnp.int32).reshape(E * EDEG)
    nn_t = node_neighbors[:, :, 0].astype(jnp.int32).T
    ne_t = node_neighbors[:, :, 1].astype(jnp.int32).T
    nn_t = jnp.pad(nn_t, ((0, 0), (0, N_PAD - N)))
    nn_t = jnp.concatenate([nn_t, jnp.arange(N_PAD, dtype=jnp.int32)[None]])
    ne_t = jnp.pad(ne_t, ((0, 0), (0, N_PAD - N)))
    nn_flat = nn_t.reshape(DEG + 1, N_CHUNKS, NCHUNK).transpose(1, 0, 2)
    nn_flat = nn_flat.reshape(N_CHUNKS * NWN)
    ne_flat = ne_t.reshape(DEG, N_CHUNKS, NCHUNK).transpose(1, 0, 2)
    ne_flat = ne_flat.reshape(N_CHUNKS * NWE)
    nf_pad = jnp.pad(node_feats, ((0, N_PAD - N), (0, 0)))

    be2 = be.reshape(1, H)
    bn2 = bn.reshape(1, H)
    bce2 = bce.reshape(1, H)
    bcn2 = bcn.reshape(1, H)
    g2 = ln_g.reshape(1, H)
    b2 = ln_b.reshape(1, H)

    agg8 = _edge_gather(edge_feats, en_flat)

    new_edge = _edge_mlp(agg8, We.T, be2, Wce.T, bce2, g2, b2)

    agg_nf, agg_ne = _node_gather(nf_pad, new_edge, nn_flat, ne_flat)

    new_node = _node_mlp(agg_nf[:N], agg_ne[:N], Wn.T, bn2, Wcn.T, bcn2,
                         g2, b2)

    return (new_node, new_edge)


# speedup vs baseline: 1.3014x; 1.3014x over previous
import functools

import jax
import jax.numpy as jnp
from jax import lax
from jax.experimental import pallas as pl
from jax.experimental.pallas import tpu as pltpu
from jax.experimental.pallas import tpu_sc as plsc

N = 10000
DEG = 16
E = 160000
EDEG = 4
DN = 128
DE = 16
H = 128

NTILES = 32

ECHUNK = 1000
E_CHUNKS_PER_TILE = E // ECHUNK // NTILES
E8 = E // 8

NCHUNK = 128
N_PAD = 10240
N_CHUNKS = N_PAD // NCHUNK

_mesh = plsc.VectorSubcoreMesh(core_axis_name="c", subcore_axis_name="s")


def _wid():
    return lax.axis_index("c") * 16 + lax.axis_index("s")


def _edge_gather_body(ef_hbm, enf_hbm, out_hbm, idxs, buf_v, own_v, acc_v,
                      isem, gsem, osem, wsem):
    wid = _wid()
    nc = E_CHUNKS_PER_TILE

    for c in range(nc):
        base = (wid * nc + c) * ECHUNK
        pltpu.async_copy(enf_hbm.at[pl.ds(base * EDEG, ECHUNK * EDEG)],
                         idxs[c], isem)

    def chunk_body(c, _):
        base = (wid * nc + c) * ECHUNK

        pltpu.make_async_copy(
            enf_hbm.at[pl.ds(0, ECHUNK * EDEG)], idxs[0], isem).wait()
        down = pltpu.async_copy(ef_hbm.at[pl.ds(base, ECHUNK)], own_v, wsem)
        for c2 in range(nc):
            @pl.when(c2 == c)
            def _fire(c2=c2):
                pltpu.async_copy(ef_hbm.at[idxs[c2]], buf_v, gsem)
        pltpu.make_async_copy(ef_hbm.at[idxs[0]], buf_v, gsem).wait()
        down.wait()

        def row_body(r, _):
            for u in range(8):
                e = r * 8 + u
                s = pl.ds(u * DE, DE)
                acc_v[r, s] = (own_v[e, :] + buf_v[4 * e, :]
                               + buf_v[4 * e + 1, :] + buf_v[4 * e + 2, :]
                               + buf_v[4 * e + 3, :])
            return _

        lax.fori_loop(0, ECHUNK // 8, row_body, None)
        pltpu.async_copy(
            acc_v, out_hbm.at[pl.ds((wid * nc + c) * (ECHUNK // 8),
                                    ECHUNK // 8)], osem)
        pltpu.make_async_copy(
            acc_v, out_hbm.at[pl.ds(0, ECHUNK // 8)], osem).wait()
        return _

    lax.fori_loop(0, nc, chunk_body, None)


@functools.partial(
    pl.kernel,
    out_type=jax.ShapeDtypeStruct((E8, 128), jnp.float32),
    mesh=_mesh,
    scratch_types=[
        [pltpu.VMEM((ECHUNK * EDEG,), jnp.int32)
         for _ in range(E_CHUNKS_PER_TILE)],
        pltpu.VMEM((ECHUNK * EDEG, DE), jnp.float32),
        pltpu.VMEM((ECHUNK, DE), jnp.float32),
        pltpu.VMEM((ECHUNK // 8, 128), jnp.float32),
        pltpu.SemaphoreType.DMA,
        pltpu.SemaphoreType.DMA,
        pltpu.SemaphoreType.DMA,
        pltpu.SemaphoreType.DMA,
    ],
    compiler_params=pltpu.CompilerParams(use_tc_tiling_on_sc=False),
)
def _edge_gather(ef_hbm, enf_hbm, out_hbm, idxs, buf_v, own_v, acc_v,
                 isem, gsem, osem, wsem):
    _edge_gather_body(ef_hbm, enf_hbm, out_hbm, idxs, buf_v, own_v, acc_v,
                      isem, gsem, osem, wsem)


NWN = (DEG + 1) * NCHUNK
NWE = DEG * NCHUNK


def _node_fire_idx(nnt, net, idxn, idxe, isem, c):
    pltpu.async_copy(nnt.at[pl.ds(c * NWN, NWN)], idxn, isem)
    pltpu.async_copy(net.at[pl.ds(c * NWE, NWE)], idxe, isem)


def _node_drain_idx(nnt, net, idxn, idxe, isem):
    pltpu.make_async_copy(nnt.at[pl.ds(0, NWN)], idxn, isem).wait()
    pltpu.make_async_copy(net.at[pl.ds(0, NWE)], idxe, isem).wait()


def _node_zero(accn, acce):
    zero16 = jnp.zeros((16,), jnp.float32)

    def zrow(r, _):
        for u in range(H // 16):
            s = pl.ds(u * 16, 16)
            accn[r, s] = zero16
            acce[r, s] = zero16
        return _

    lax.fori_loop(0, NCHUNK, zrow, None)


def _node_fire_adds(nf, ne_tab, idxn, idxe, accn, acce, gsem):
    for j in range(DEG + 1):
        pltpu.async_copy(nf.at[idxn.at[pl.ds(j * NCHUNK, NCHUNK)]], accn,
                         gsem, add=True)
    for j in range(DEG):
        pltpu.async_copy(ne_tab.at[idxe.at[pl.ds(j * NCHUNK, NCHUNK)]], acce,
                         gsem, add=True)


def _node_drain_adds(nf, ne_tab, idxn, idxe, accn, acce, gsem):
    for _ in range(DEG + 1):
        pltpu.make_async_copy(
            nf.at[idxn.at[pl.ds(0, NCHUNK)]], accn, gsem).wait()
    for _ in range(DEG):
        pltpu.make_async_copy(
            ne_tab.at[idxe.at[pl.ds(0, NCHUNK)]], acce, gsem).wait()


def _node_fire_outs(accn, acce, outn, oute, osem, base):
    pltpu.async_copy(accn, outn.at[pl.ds(base, NCHUNK)], osem)
    pltpu.async_copy(acce, oute.at[pl.ds(base, NCHUNK)], osem)


def _node_drain_outs(accn, acce, outn, oute, osem):
    pltpu.make_async_copy(accn, outn.at[pl.ds(0, NCHUNK)], osem).wait()
    pltpu.make_async_copy(acce, oute.at[pl.ds(0, NCHUNK)], osem).wait()


def _node_gather_body(nf, ne_tab, nnt, net, outn, oute,
                      idxns, idxes, accns, acces, isems, gsems, osems):
    wid = _wid()
    c0 = wid * 2
    c1 = wid * 2 + 1
    c2 = 64 + (wid // 16) * 8 + (wid % 16)
    has3 = (wid % 16) < 8

    _node_fire_idx(nnt, net, idxns[0], idxes[0], isems[0], c0)
    _node_fire_idx(nnt, net, idxns[1], idxes[1], isems[1], c1)
    _node_zero(accns[0], acces[0])
    _node_drain_idx(nnt, net, idxns[0], idxes[0], isems[0])
    _node_fire_adds(nf, ne_tab, idxns[0], idxes[0], accns[0], acces[0],
                    gsems[0])
    _node_zero(accns[1], acces[1])
    _node_drain_idx(nnt, net, idxns[1], idxes[1], isems[1])
    _node_fire_adds(nf, ne_tab, idxns[1], idxes[1], accns[1], acces[1],
                    gsems[1])

    _node_drain_adds(nf, ne_tab, idxns[0], idxes[0], accns[0], acces[0],
                     gsems[0])
    _node_fire_outs(accns[0], acces[0], outn, oute, osems[0], c0 * NCHUNK)

    @pl.when(has3)
    def _start_c2():
        _node_fire_idx(nnt, net, idxns[2], idxes[2], isems[0], c2)
        _node_drain_outs(accns[0], acces[0], outn, oute, osems[0])
        _node_zero(accns[0], acces[0])
        _node_drain_idx(nnt, net, idxns[2], idxes[2], isems[0])
        _node_fire_adds(nf, ne_tab, idxns[2], idxes[2], accns[0], acces[0],
                        gsems[0])

    _node_drain_adds(nf, ne_tab, idxns[1], idxes[1], accns[1], acces[1],
                     gsems[1])
    _node_fire_outs(accns[1], acces[1], outn, oute, osems[1], c1 * NCHUNK)

    @pl.when(has3)
    def _end_c2():
        _node_drain_adds(nf, ne_tab, idxns[2], idxes[2], accns[0], acces[0],
                         gsems[0])
        _node_fire_outs(accns[0], acces[0], outn, oute, osems[0], c2 * NCHUNK)
        _node_drain_outs(accns[0], acces[0], outn, oute, osems[0])

    _node_drain_outs(accns[1], acces[1], outn, oute, osems[1])


@functools.partial(
    pl.kernel,
    out_type=(jax.ShapeDtypeStruct((N_PAD, DN), jnp.float32),
              jax.ShapeDtypeStruct((N_PAD, H), jnp.float32)),
    mesh=_mesh,
    scratch_types=[
        [pltpu.VMEM((NWN,), jnp.int32) for _ in range(3)],
        [pltpu.VMEM((NWE,), jnp.int32) for _ in range(3)],
        [pltpu.VMEM((NCHUNK, DN), jnp.float32) for _ in range(2)],
        [pltpu.VMEM((NCHUNK, H), jnp.float32) for _ in range(2)],
        [pltpu.SemaphoreType.DMA for _ in range(2)],
        [pltpu.SemaphoreType.DMA for _ in range(2)],
        [pltpu.SemaphoreType.DMA for _ in range(2)],
    ],
)
def _node_gather(nf, ne_tab, nnt, net, outn, oute,
                 idxns, idxes, accns, acces, isems, gsems, osems):
    _node_gather_body(nf, ne_tab, nnt, net, outn, oute,
                      idxns, idxes, accns, acces, isems, gsems, osems)


def _edge_mlp_kernel(x8_ref, w1_ref, b1_ref, w2_ref, b2_ref, g_ref, b_ref,
                     o_ref):
    w1 = w1_ref[...]
    w2 = w2_ref[...]
    b1 = float(EDEG + 1) * b1_ref[...]
    b2 = b2_ref[...]
    g = g_ref[...]
    b = b_ref[...]
    for u in range(8):
        x = x8_ref[:, u * DE:(u + 1) * DE]
        h1 = jnp.dot(x, w1, preferred_element_type=jnp.float32) + b1
        h2 = jax.nn.relu(
            jnp.dot(h1.astype(jnp.bfloat16), w2.astype(jnp.bfloat16),
                    preferred_element_type=jnp.float32) + b2)
        m = jnp.mean(h2, axis=-1, keepdims=True)
        v = jnp.mean((h2 - m) ** 2, axis=-1, keepdims=True)
        o_ref[:, u, :] = (h2 - m) / jnp.sqrt(v + 1e-5) * g + b


def _edge_mlp(agg8, WeT, be2, WceT, bce2, g2, b2):
    blk = 1000
    grid = E8 // blk
    out3 = pl.pallas_call(
        _edge_mlp_kernel,
        grid=(grid,),
        in_specs=[
            pl.BlockSpec((blk, 128), lambda i: (i, 0)),
            pl.BlockSpec((DE, H), lambda i: (0, 0)),
            pl.BlockSpec((1, H), lambda i: (0, 0)),
            pl.BlockSpec((H, H), lambda i: (0, 0)),
            pl.BlockSpec((1, H), lambda i: (0, 0)),
            pl.BlockSpec((1, H), lambda i: (0, 0)),
            pl.BlockSpec((1, H), lambda i: (0, 0)),
        ],
        out_specs=pl.BlockSpec((blk, 8, H), lambda i: (i, 0, 0)),
        out_shape=jax.ShapeDtypeStruct((E8, 8, H), jnp.float32),
    )(agg8, WeT, be2, WceT, bce2, g2, b2)
    return out3.reshape(E, H)


def _node_mlp_kernel(x_ref, e_ref, w1_ref, b1_ref, w2_ref, b2_ref, g_ref,
                     b_ref, o_ref):
    h1 = (jnp.dot(x_ref[...], w1_ref[...], preferred_element_type=jnp.float32)
          + float(DEG + 1) * b1_ref[...] + e_ref[...])
    h2 = jax.nn.relu(jnp.dot(h1, w2_ref[...],
                             preferred_element_type=jnp.float32) + b2_ref[...])
    m = jnp.mean(h2, axis=-1, keepdims=True)
    v = jnp.mean((h2 - m) ** 2, axis=-1, keepdims=True)
    o_ref[...] = (h2 - m) / jnp.sqrt(v + 1e-5) * g_ref[...] + b_ref[...]


def _node_mlp(agg_nf, agg_ne, WnT, bn2, WcnT, bcn2, g2, b2):
    blk = 1000
    grid = N // blk
    return pl.pallas_call(
        _node_mlp_kernel,
        grid=(grid,),
        in_specs=[
            pl.BlockSpec((blk, DN), lambda i: (i, 0)),
            pl.BlockSpec((blk, H), lambda i: (i, 0)),
            pl.BlockSpec((DN, H), lambda i: (0, 0)),
            pl.BlockSpec((1, H), lambda i: (0, 0)),
            pl.BlockSpec((H, H), lambda i: (0, 0)),
            pl.BlockSpec((1, H), lambda i: (0, 0)),
            pl.BlockSpec((1, H), lambda i: (0, 0)),
            pl.BlockSpec((1, H), lambda i: (0, 0)),
        ],
        out_specs=pl.BlockSpec((blk, H), lambda i: (i, 0)),
        out_shape=jax.ShapeDtypeStruct((N, H), jnp.float32),
    )(agg_nf, agg_ne, WnT, bn2, WcnT, bcn2, g2, b2)


def kernel(node_neighbors, edge_neighbors, node_feats, edge_feats,
           We, be, Wn, bn, Wce, bce, Wcn, bcn, ln_g, ln_b):
    en_flat = edge_neighbors.astype(jnp.int32).reshape(E * EDEG)
    nn_t = node_neighbors[:, :, 0].astype(jnp.int32).T
    ne_t = node_neighbors[:, :, 1].astype(jnp.int32).T
    nn_t = jnp.pad(nn_t, ((0, 0), (0, N_PAD - N)))
    nn_t = jnp.concatenate([nn_t, jnp.arange(N_PAD, dtype=jnp.int32)[None]])
    ne_t = jnp.pad(ne_t, ((0, 0), (0, N_PAD - N)))
    c_ids = jnp.arange(N_CHUNKS, dtype=jnp.int32)
    sc1 = jnp.where(c_ids < 64, (c_ids // 2) // 16, (c_ids - 64) // 8)
    off = (sc1 * N_PAD).astype(jnp.int32)[None, :, None]
    nn_flat = nn_t.reshape(DEG + 1, N_CHUNKS, NCHUNK) + off
    nn_flat = nn_flat.transpose(1, 0, 2).reshape(N_CHUNKS * NWN)
    ne_flat = ne_t.reshape(DEG, N_CHUNKS, NCHUNK) + off
    ne_flat = ne_flat.transpose(1, 0, 2).reshape(N_CHUNKS * NWE)
    nf_pad = jnp.pad(node_feats, ((0, N_PAD - N), (0, 0)))
    nf2 = jnp.concatenate([nf_pad, nf_pad])

    be2 = be.reshape(1, H)
    bn2 = bn.reshape(1, H)
    bce2 = bce.reshape(1, H)
    bcn2 = bcn.reshape(1, H)
    g2 = ln_g.reshape(1, H)
    b2 = ln_b.reshape(1, H)

    agg8 = _edge_gather(edge_feats, en_flat)

    new_edge = _edge_mlp(agg8, We.T, be2, Wce.T, bce2, g2, b2)

    ne_tab2 = jnp.concatenate([new_edge[:N_PAD], new_edge[:N_PAD]])
    agg_nf, agg_ne = _node_gather(nf2, ne_tab2, nn_flat, ne_flat)

    new_node = _node_mlp(agg_nf[:N], agg_ne[:N], Wn.T, bn2, Wcn.T, bcn2,
                         g2, b2)

    return (new_node, new_edge)
